# baseline (speedup 1.0000x reference)
import functools

import jax
import jax.numpy as jnp
from jax import lax
from jax.experimental import pallas as pl
from jax.experimental.pallas import tpu as pltpu
from jax.experimental.pallas import tpu_sc as plsc

_BM = 2048


def _make_gather(F, V, D, B):
    info = plsc.get_sparse_core_info()
    NC, NS = info.num_cores, info.num_subcores
    NW = NC * NS
    assert D == NW
    FD = F * D

    M = 49920
    W0 = (V // 128) * 128
    RM = W0 - M
    TAIL = V - W0
    PH = 4096
    NPH = B // PH
    mesh = plsc.VectorSubcoreMesh(core_axis_name="c", subcore_axis_name="s")

    @functools.partial(
        pl.kernel,
        mesh=mesh,
        out_type=jax.ShapeDtypeStruct((FD, B), jnp.float32),
        scratch_types=[
            pltpu.VMEM((W0,), jnp.float32),
            pltpu.VMEM((TAIL,), jnp.float32),
            pltpu.VMEM((PH,), jnp.int32),
            pltpu.VMEM((PH,), jnp.int32),
            pltpu.VMEM((PH,), jnp.float32),
            pltpu.VMEM((PH,), jnp.float32),
            pltpu.SemaphoreType.DMA,
            pltpu.SemaphoreType.DMA,
            pltpu.SemaphoreType.DMA,
            pltpu.SemaphoreType.DMA,
            pltpu.SemaphoreType.DMA,
            pltpu.SemaphoreType.DMA,
            pltpu.SemaphoreType.DMA,
        ],
        compiler_params=pltpu.CompilerParams(
            use_tc_tiling_on_sc=True, needs_layout_passes=False),
    )
    def gather(tab_hbm, idx_hbm, out_hbm, row_v, tail_v, ia, ib, oa, ob,
               semL, semR, semT, si0, si1, sf0, sf1):
        w = lax.axis_index("s") * NC + lax.axis_index("c")
        iota = lax.iota(jnp.int32, 16)
        ibufs = [(ia, si0), (ib, si1)]
        obufs = [(oa, sf0), (ob, sf1)]

        def Ldesc(i):
            fd = i * D + w
            return pltpu.make_async_copy(
                tab_hbm.at[fd // 8, fd % 8, pl.ds(0, M)],
                row_v.at[pl.ds(0, M)], semL)

        def Rdesc(i):
            fd = i * D + w
            return pltpu.make_async_copy(
                tab_hbm.at[fd // 8, fd % 8, pl.ds(M, RM)],
                row_v.at[pl.ds(M, RM)], semR)

        def Tdesc(i):
            fd = i * D + w
            return pltpu.make_async_copy(
                tab_hbm.at[fd // 8, fd % 8, pl.ds(W0, TAIL)], tail_v, semT)

        def Idesc(i, p, buf, sem):
            return pltpu.make_async_copy(
                idx_hbm.at[i, pl.ds(p * PH, PH)], buf, sem)

        def Odesc(i, p, buf, sem):
            return pltpu.make_async_copy(
                buf, out_hbm.at[i * D + w, pl.ds(p * PH, PH)], sem)

        Ldesc(0).start()
        Idesc(0, 0, ia, si0).start()

        def field(i, carry):
            Rdesc(i).start()
            Tdesc(i).start()
            Ldesc(i).wait()
            for p in range(NPH):
                ibuf, isem = ibufs[p % 2]
                obuf, osem = obufs[p % 2]
                Idesc(i, p, ibuf, isem).wait()
                if p < NPH - 1:
                    nb, ns = ibufs[(p + 1) % 2]
                    Idesc(i, p + 1, nb, ns).start()
                else:
                    @pl.when(i + 1 < F)
                    def _():
                        Idesc(i + 1, 0, ia, si0).start()
                if p >= 2:
                    Odesc(i, p - 2, obuf, osem).wait()

                @plsc.parallel_loop(0, PH, 16, unroll=16)
                def passL(o):
                    iv = ibuf[pl.ds(o, 16)]
                    pcl = jnp.minimum(iv, W0 - 1)
                    obuf[pl.ds(o, 16)] = plsc.load_gather(row_v, [pcl])

                if p == 0:
                    Rdesc(i).wait()
                    Tdesc(i).wait()
                if p == NPH - 1:
                    @pl.when(i + 1 < F)
                    def _():
                        Ldesc(i + 1).start()

                @plsc.parallel_loop(0, PH, 16, unroll=16)
                def passR(o):
                    iv = ibuf[pl.ds(o, 16)]
                    pm = jnp.minimum(jnp.maximum(iv, M), W0 - 1)
                    gm = plsc.load_gather(row_v, [pm])
                    pt = jnp.maximum(iv - W0, 0)
                    gt = plsc.load_gather(tail_v, [pt])
                    v = jnp.where(iv >= W0, gt, gm)
                    plsc.store_scatter(obuf, [o + iota], v, mask=iv >= M)

                Odesc(i, p, obuf, osem).start()
            Odesc(i, NPH - 2, obufs[(NPH - 2) % 2][0], obufs[(NPH - 2) % 2][1]).wait()
            Odesc(i, NPH - 1, obufs[(NPH - 1) % 2][0], obufs[(NPH - 1) % 2][1]).wait()
            return carry

        lax.fori_loop(0, F, field, 0)

    return gather


def _mlp(emb_t, num_t, W1e, W1n, b1, W2, b2, W3, b3, W4, b4):
    FD, Bt = emb_t.shape
    NUM = num_t.shape[0]
    cdim0 = (((0,), (0,)), ((), ()))

    def body(emb_ref, num_ref, w1e_ref, w1n_ref, b1_ref, w2_ref, b2_ref,
             w3_ref, b3_ref, w4_ref, b4_ref, out_ref):
        bf = jnp.bfloat16
        h = lax.dot_general(emb_ref[...].astype(bf), w1e_ref[...].astype(bf), cdim0,
                            preferred_element_type=jnp.float32)
        h = h + lax.dot_general(num_ref[...], w1n_ref[...], cdim0,
                                preferred_element_type=jnp.float32)
        h = jnp.maximum(h + b1_ref[...], 0.0)
        h = jnp.maximum(jnp.dot(h.astype(bf), w2_ref[...].astype(bf), preferred_element_type=jnp.float32) + b2_ref[...], 0.0)
        h = jnp.maximum(jnp.dot(h.astype(bf), w3_ref[...].astype(bf), preferred_element_type=jnp.float32) + b3_ref[...], 0.0)
        out_ref[...] = lax.dot_general(w4_ref[...], h, (((0,), (1,)), ((), ())),
                                       preferred_element_type=jnp.float32) + b4_ref[...]

    def full(a):
        nd = a.ndim
        return pl.BlockSpec(a.shape, lambda i, _nd=nd: (0,) * _nd)

    return pl.pallas_call(
        body,
        grid=(Bt // _BM,),
        in_specs=[
            pl.BlockSpec((FD, _BM), lambda i: (0, i)),
            pl.BlockSpec((NUM, _BM), lambda i: (0, i)),
            full(W1e), full(W1n), full(b1),
            full(W2), full(b2), full(W3), full(b3), full(W4), full(b4),
        ],
        out_specs=pl.BlockSpec((1, _BM), lambda i: (0, i)),
        out_shape=jax.ShapeDtypeStruct((1, Bt), jnp.float32),
    )(emb_t, num_t, W1e, W1n, b1, W2, b2, W3, b3, W4, b4)


def kernel(categorical_inputs, numeric_inputs, tables, W1, b1, W2, b2, W3, b3, W4, b4):
    B, F = categorical_inputs.shape
    _, V, D = tables.shape
    FD = F * D

    tab_rows = tables.transpose(0, 2, 1).reshape(FD // 8, 8, V)
    idx_t = categorical_inputs.T
    num_t = numeric_inputs.T

    emb_t = _make_gather(F, V, D, B)(tab_rows, idx_t)

    out = _mlp(
        emb_t, num_t,
        W1[:FD], W1[FD:], b1.reshape(1, -1),
        W2, b2.reshape(1, -1), W3, b3.reshape(1, -1), W4, b4.reshape(1, -1),
    )
    return out.reshape(B)


# speedup vs baseline: 1.0699x; 1.0699x over previous
import functools

import jax
import jax.numpy as jnp
from jax import lax
from jax.experimental import pallas as pl
from jax.experimental.pallas import tpu as pltpu
from jax.experimental.pallas import tpu_sc as plsc

_BM = 2048


def _make_gather(F, V, D, B):
    info = plsc.get_sparse_core_info()
    NC, NS = info.num_cores, info.num_subcores
    NW = NC * NS
    assert D == NW
    FD = F * D

    M = 49920
    W0 = (V // 128) * 128
    RM = W0 - M
    TAIL = V - W0
    PH = 4096
    NPH = B // PH
    mesh = plsc.VectorSubcoreMesh(core_axis_name="c", subcore_axis_name="s")

    @functools.partial(
        pl.kernel,
        mesh=mesh,
        out_type=jax.ShapeDtypeStruct((FD, B), jnp.float32),
        scratch_types=[
            pltpu.VMEM((W0,), jnp.float32),
            pltpu.VMEM((TAIL,), jnp.float32),
            pltpu.VMEM((PH,), jnp.int32),
            pltpu.VMEM((PH,), jnp.int32),
            pltpu.VMEM((PH,), jnp.float32),
            pltpu.VMEM((PH,), jnp.float32),
            pltpu.SemaphoreType.DMA,
            pltpu.SemaphoreType.DMA,
            pltpu.SemaphoreType.DMA,
            pltpu.SemaphoreType.DMA,
            pltpu.SemaphoreType.DMA,
            pltpu.SemaphoreType.DMA,
            pltpu.SemaphoreType.DMA,
        ],
        compiler_params=pltpu.CompilerParams(
            use_tc_tiling_on_sc=True, needs_layout_passes=False),
    )
    def gather(tab_hbm, idx_hbm, out_hbm, row_v, tail_v, ia, ib, oa, ob,
               semL, semR, semT, si0, si1, sf0, sf1):
        w = lax.axis_index("s") * NC + lax.axis_index("c")
        iota = lax.iota(jnp.int32, 16)
        ibufs = [(ia, si0), (ib, si1)]
        obufs = [(oa, sf0), (ob, sf1)]

        def Ldesc(i):
            fd = i * D + w
            return pltpu.make_async_copy(
                tab_hbm.at[fd // 8, fd % 8, pl.ds(0, M)],
                row_v.at[pl.ds(0, M)], semL)

        def Rdesc(i):
            fd = i * D + w
            return pltpu.make_async_copy(
                tab_hbm.at[fd // 8, fd % 8, pl.ds(M, RM)],
                row_v.at[pl.ds(M, RM)], semR)

        def Tdesc(i):
            fd = i * D + w
            return pltpu.make_async_copy(
                tab_hbm.at[fd // 8, fd % 8, pl.ds(W0, TAIL)], tail_v, semT)

        def Idesc(i, p, buf, sem):
            return pltpu.make_async_copy(
                idx_hbm.at[i, pl.ds(p * PH, PH)], buf, sem)

        def Odesc(i, p, buf, sem):
            return pltpu.make_async_copy(
                buf, out_hbm.at[i * D + w, pl.ds(p * PH, PH)], sem)

        Ldesc(0).start()
        Idesc(0, 0, ia, si0).start()

        def field(i, carry):
            Rdesc(i).start()
            Tdesc(i).start()
            Ldesc(i).wait()
            for p in range(NPH):
                ibuf, isem = ibufs[p % 2]
                obuf, osem = obufs[p % 2]
                Idesc(i, p, ibuf, isem).wait()
                if p < NPH - 1:
                    nb, ns = ibufs[(p + 1) % 2]
                    Idesc(i, p + 1, nb, ns).start()
                else:
                    @pl.when(i + 1 < F)
                    def _():
                        Idesc(i + 1, 0, ia, si0).start()
                if p >= 2:
                    Odesc(i, p - 2, obuf, osem).wait()

                @plsc.parallel_loop(0, PH, 16, unroll=8)
                def passL(o):
                    iv = ibuf[pl.ds(o, 16)]
                    pcl = jnp.minimum(iv, W0 - 1)
                    obuf[pl.ds(o, 16)] = plsc.load_gather(row_v, [pcl])

                if p == 0:
                    Rdesc(i).wait()
                    Tdesc(i).wait()
                if p == NPH - 1:
                    @pl.when(i + 1 < F)
                    def _():
                        Ldesc(i + 1).start()

                @plsc.parallel_loop(0, PH, 16, unroll=8)
                def passR(o):
                    iv = ibuf[pl.ds(o, 16)]
                    pm = jnp.minimum(jnp.maximum(iv, M), W0 - 1)
                    gm = plsc.load_gather(row_v, [pm])
                    pt = jnp.maximum(iv - W0, 0)
                    gt = plsc.load_gather(tail_v, [pt])
                    v = jnp.where(iv >= W0, gt, gm)
                    plsc.store_scatter(obuf, [o + iota], v, mask=iv >= M)

                Odesc(i, p, obuf, osem).start()
            Odesc(i, NPH - 2, obufs[(NPH - 2) % 2][0], obufs[(NPH - 2) % 2][1]).wait()
            Odesc(i, NPH - 1, obufs[(NPH - 1) % 2][0], obufs[(NPH - 1) % 2][1]).wait()
            return carry

        lax.fori_loop(0, F, field, 0)

    return gather


def _mlp(emb_t, num_t, W1e, W1n, b1, W2, b2, W3, b3, W4, b4):
    FD, Bt = emb_t.shape
    NUM = num_t.shape[0]
    cdim0 = (((0,), (0,)), ((), ()))

    def body(emb_ref, num_ref, w1e_ref, w1n_ref, b1_ref, w2_ref, b2_ref,
             w3_ref, b3_ref, w4_ref, b4_ref, out_ref):
        bf = jnp.bfloat16
        h = lax.dot_general(emb_ref[...].astype(bf), w1e_ref[...].astype(bf), cdim0,
                            preferred_element_type=jnp.float32)
        h = h + lax.dot_general(num_ref[...], w1n_ref[...], cdim0,
                                preferred_element_type=jnp.float32)
        h = jnp.maximum(h + b1_ref[...], 0.0)
        h = jnp.maximum(jnp.dot(h.astype(bf), w2_ref[...].astype(bf), preferred_element_type=jnp.float32) + b2_ref[...], 0.0)
        h = jnp.maximum(jnp.dot(h.astype(bf), w3_ref[...].astype(bf), preferred_element_type=jnp.float32) + b3_ref[...], 0.0)
        out_ref[...] = lax.dot_general(w4_ref[...], h, (((0,), (1,)), ((), ())),
                                       preferred_element_type=jnp.float32) + b4_ref[...]

    def full(a):
        nd = a.ndim
        return pl.BlockSpec(a.shape, lambda i, _nd=nd: (0,) * _nd)

    return pl.pallas_call(
        body,
        grid=(Bt // _BM,),
        in_specs=[
            pl.BlockSpec((FD, _BM), lambda i: (0, i)),
            pl.BlockSpec((NUM, _BM), lambda i: (0, i)),
            full(W1e), full(W1n), full(b1),
            full(W2), full(b2), full(W3), full(b3), full(W4), full(b4),
        ],
        out_specs=pl.BlockSpec((1, _BM), lambda i: (0, i)),
        out_shape=jax.ShapeDtypeStruct((1, Bt), jnp.float32),
    )(emb_t, num_t, W1e, W1n, b1, W2, b2, W3, b3, W4, b4)


def kernel(categorical_inputs, numeric_inputs, tables, W1, b1, W2, b2, W3, b3, W4, b4):
    B, F = categorical_inputs.shape
    _, V, D = tables.shape
    FD = F * D

    tab_rows = tables.transpose(0, 2, 1).reshape(FD // 8, 8, V)
    idx_t = categorical_inputs.T
    num_t = numeric_inputs.T

    emb_t = _make_gather(F, V, D, B)(tab_rows, idx_t)

    out = _mlp(
        emb_t, num_t,
        W1[:FD], W1[FD:], b1.reshape(1, -1),
        W2, b2.reshape(1, -1), W3, b3.reshape(1, -1), W4, b4.reshape(1, -1),
    )
    return out.reshape(B)
